# baseline (speedup 1.0000x reference)
import functools

import jax
import jax.numpy as jnp
from jax import lax
from jax.experimental import pallas as pl
from jax.experimental.pallas import tpu as pltpu
from jax.experimental.pallas import tpu_sc as plsc

N_FIELDS = 26
VOCAB = 100000
DIM = 32
B = 1024
T = 50

BLK_B = 128
NBLK_B = B // BLK_B
NW = 32
BLK_W = DIM * BLK_B
N_GROUPS = 1
F_G = N_FIELDS // N_GROUPS

_mesh = plsc.VectorSubcoreMesh(core_axis_name="c", subcore_axis_name="s")


def _make_gather(n_fields):
    n_blocks = n_fields * T * NBLK_B
    per_w = (n_blocks + NW - 1) // NW
    nd = 4
    n_quad = (per_w + nd - 1 + (nd - 1)) // nd

    @functools.partial(
        pl.kernel,
        mesh=_mesh,
        out_type=jax.ShapeDtypeStruct((n_fields * T * DIM * B,), jnp.float32),
        scratch_types=(
            [pltpu.VMEM((BLK_B,), jnp.int32)] * nd
            + [pltpu.VMEM((BLK_B, DIM), jnp.float32)] * nd
            + [pltpu.VMEM((BLK_W,), jnp.float32)] * nd
            + [pltpu.SemaphoreType.DMA] * (3 * nd)
        ),
        compiler_params=pltpu.CompilerParams(
            use_tc_tiling_on_sc=False, needs_layout_passes=False
        ),
    )
    def sc_gather(idx_hbm, tab_hbm, out_hbm, *bufs):
        cid = lax.axis_index("c")
        sid = lax.axis_index("s")
        wid = sid * 2 + cid
        base = wid * per_w
        last = jnp.minimum(base + per_w - 1, n_blocks - 1)
        base = jnp.minimum(base, last)

        idx_v = list(bufs[0:nd])
        gath_v = list(bufs[nd:2 * nd])
        tr_v = list(bufs[2 * nd:3 * nd])
        isem = list(bufs[3 * nd:4 * nd])
        gsem = list(bufs[4 * nd:5 * nd])
        osem = list(bufs[5 * nd:6 * nd])

        scat = [lax.iota(jnp.int32, 16) * BLK_B + h * 16 * BLK_B
                for h in range(2)]

        def fire_gather(u, slot):
            f = u // (T * NBLK_B)
            pltpu.async_copy(tab_hbm.at[f].at[idx_v[slot]], gath_v[slot],
                             gsem[slot])

        def wait_gather(slot):
            pltpu.make_async_copy(tab_hbm.at[0].at[pl.ds(0, BLK_B), :],
                                  gath_v[slot], gsem[slot]).wait()

        def fire_idx(u, slot):
            pltpu.async_copy(idx_hbm.at[pl.ds(u * BLK_B, BLK_B)], idx_v[slot],
                             isem[slot])

        def wait_idx(slot):
            pltpu.make_async_copy(idx_hbm.at[pl.ds(0, BLK_B)], idx_v[slot],
                                  isem[slot]).wait()

        def wait_outs(slot):
            pltpu.make_async_copy(out_hbm.at[pl.ds(0, BLK_W)], tr_v[slot],
                                  osem[slot]).wait()

        def do_slot(m, s, j):
            u = jnp.minimum(base + s, last)
            ug = jnp.minimum(base + s + nd - 1, last)
            up = jnp.minimum(base + s + nd, last)
            wait_gather(j)
            fire_idx(up, j)
            jg = (j + nd - 1) % nd
            wait_idx(jg)
            fire_gather(ug, jg)
            @pl.when(m > 0)
            def _():
                wait_outs(j)
            prev = None
            for row in range(BLK_B):
                cur = [gath_v[j][row, pl.ds(h * 16, 16)] for h in range(2)]
                if prev is not None:
                    for h in range(2):
                        plsc.store_scatter(tr_v[j], [scat[h] + (row - 1)],
                                           prev[h])
                prev = cur
            for h in range(2):
                plsc.store_scatter(tr_v[j], [scat[h] + (BLK_B - 1)], prev[h])
            ft = u // NBLK_B
            bc = u % NBLK_B
            out_base = ft * (DIM * B) + bc * (8 * BLK_B)
            for dt in range(4):
                pltpu.async_copy(
                    tr_v[j].at[pl.ds(dt * 8 * BLK_B, 8 * BLK_B)],
                    out_hbm.at[pl.ds(out_base + dt * (8 * B), 8 * BLK_B)],
                    osem[j],
                )

        def quad_body(m, carry):
            for j in range(nd):
                do_slot(m, nd * m + j, j)
            return carry

        for j in range(nd - 1):
            pltpu.sync_copy(
                idx_hbm.at[pl.ds(jnp.minimum(base + j, last) * BLK_B, BLK_B)],
                idx_v[j])
            fire_gather(jnp.minimum(base + j, last), j)
        fire_idx(jnp.minimum(base + nd - 1, last), nd - 1)
        lax.fori_loop(0, n_quad, quad_body, 0)
        for j in range(nd - 1):
            wait_gather(j)
        wait_idx(nd - 1)
        for j in range(nd):
            wait_outs(j)

    return sc_gather


_gather_g = _make_gather(F_G)


def kernel(x, tables):
    flat_idx = x.transpose(2, 1, 0).reshape(N_FIELDS * T * B)
    tabw = jax.lax.optimization_barrier(
        tables.reshape(N_FIELDS * VOCAB * DIM // 128, 128)
    )
    out = _gather_g(flat_idx, tabw.reshape(N_FIELDS, VOCAB, DIM))
    out = out.reshape(N_FIELDS, T, DIM // 8, B // BLK_B, 8, BLK_B)
    out = out.transpose(0, 3, 5, 1, 2, 4).reshape(N_FIELDS, B, T, DIM)
    return out
